# baseline (speedup 1.0000x reference)
import functools

import jax
import jax.numpy as jnp
from jax import lax
from jax.experimental import pallas as pl
from jax.experimental.pallas import tpu as pltpu
from jax.experimental.pallas import tpu_sc as plsc

BATCH = 4
SEQ = 8192
DIM = 1024
CHUNK = 32


def _pos_embed_kernel(table_hbm, out_hbm, buf0, buf1, rs0, rs1, ws0, ws1):
    info = plsc.get_sparse_core_info()
    nc, ns = info.num_cores, info.num_subcores
    nw = nc * ns
    rows_per_w = SEQ // nw
    wid = lax.axis_index("s") * nc + lax.axis_index("c")
    base = wid * rows_per_w
    nchunks = rows_per_w // CHUNK

    bufs = (buf0, buf1)
    rsems = (rs0, rs1)
    wsems = (ws0, ws1)

    def read(i):
        row = base + i * CHUNK
        return pltpu.async_copy(table_hbm.at[pl.ds(row, CHUNK)], bufs[i % 2],
                                rsems[i % 2])

    def write(i, b):
        row = base + i * CHUNK
        return pltpu.async_copy(bufs[i % 2], out_hbm.at[b, pl.ds(row, CHUNK)],
                                wsems[i % 2])

    rd = read(0)
    pending = [None, None]
    for i in range(nchunks):
        if i + 1 < nchunks:
            if pending[(i + 1) % 2] is not None:
                for h in pending[(i + 1) % 2]:
                    h.wait()
            nxt = read(i + 1)
        rd.wait()
        pending[i % 2] = [write(i, b) for b in range(BATCH)]
        if i + 1 < nchunks:
            rd = nxt
    for hs in pending:
        if hs is not None:
            for h in hs:
                h.wait()


@jax.jit
def _pos_embed(table):
    mesh = plsc.VectorSubcoreMesh(core_axis_name="c", subcore_axis_name="s")
    fn = functools.partial(
        pl.kernel,
        mesh=mesh,
        out_type=jax.ShapeDtypeStruct((BATCH, SEQ, DIM), jnp.float32),
        scratch_types=[
            pltpu.VMEM((CHUNK, DIM), jnp.float32),
            pltpu.VMEM((CHUNK, DIM), jnp.float32),
            pltpu.SemaphoreType.DMA,
            pltpu.SemaphoreType.DMA,
            pltpu.SemaphoreType.DMA,
            pltpu.SemaphoreType.DMA,
        ],
    )(_pos_embed_kernel)
    return fn(table)


def kernel(inputs, table):
    del inputs
    return _pos_embed(table)
